# Initial kernel scaffold; baseline (speedup 1.0000x reference)
import jax
import jax.numpy as jnp
from jax.experimental import pallas as pl


def kernel(x, table, pe):
    raise NotImplementedError("write your pallas kernel here")


# speedup vs baseline: 1.0893x; 1.0893x over previous
import functools

import jax
import jax.numpy as jnp
from jax import lax
from jax.experimental import pallas as pl
from jax.experimental.pallas import tpu as pltpu
from jax.experimental.pallas import tpu_sc as plsc

NC, NS = 2, 16
NW = NC * NS
LANES = 16
CHUNK = 16


@functools.cache
def _build(tok, seq, vocab, d):
    tpw = tok // NW
    nch = tpw // CHUNK
    vecs = d // LANES
    mesh = plsc.VectorSubcoreMesh(core_axis_name="c", subcore_axis_name="s")

    @functools.partial(
        pl.kernel,
        mesh=mesh,
        out_type=jax.ShapeDtypeStruct((tok, d), jnp.float32),
        scratch_types=[
            pltpu.VMEM((tpw,), jnp.int32),
            pltpu.VMEM((CHUNK, d), jnp.float32),
            pltpu.VMEM((CHUNK, d), jnp.float32),
            pltpu.VMEM((CHUNK, d), jnp.float32),
            pltpu.VMEM((CHUNK, d), jnp.float32),
            pltpu.SemaphoreType.DMA,
            pltpu.SemaphoreType.DMA,
            pltpu.SemaphoreType.DMA,
            pltpu.SemaphoreType.DMA,
            pltpu.SemaphoreType.DMA,
            pltpu.SemaphoreType.DMA,
        ],
    )
    def emb(x_hbm, table_hbm, pe_hbm, out_hbm,
            idx_v, g0, g1, p0, p1, sg0, sg1, sp0, sp1, ss0, ss1):
        wid = lax.axis_index("s") * NC + lax.axis_index("c")
        base = wid * tpw
        pbase = lax.rem(base, seq)
        gbuf = [g0, g1]
        pbuf = [p0, p1]
        sg = [sg0, sg1]
        sp = [sp0, sp1]
        ss = [ss0, ss1]
        gd = [None] * nch
        pd = [None] * nch
        sd = [None] * nch

        pltpu.sync_copy(x_hbm.at[pl.ds(base, tpw)], idx_v)

        def start(c):
            slot = c % 2
            off = c * CHUNK
            pd[c] = pltpu.async_copy(
                pe_hbm.at[pl.ds(pbase + off, CHUNK)], pbuf[slot], sp[slot])
            gd[c] = pltpu.async_copy(
                table_hbm.at[idx_v.at[pl.ds(off, CHUNK)]], gbuf[slot], sg[slot])

        def finish(c):
            slot = c % 2
            gd[c].wait()
            pd[c].wait()
            g, p = gbuf[slot], pbuf[slot]

            def row(r, _):
                for j in range(vecs):
                    plsc.addupdate(g.at[r, pl.ds(j * LANES, LANES)],
                                   p[r, pl.ds(j * LANES, LANES)])
                return 0

            lax.fori_loop(0, CHUNK, row, 0, unroll=False)
            sd[c] = pltpu.async_copy(
                g, out_hbm.at[pl.ds(base + c * CHUNK, CHUNK)], ss[slot])

        for c in range(nch):
            if c >= 2:
                sd[c - 2].wait()
            start(c)
            if c >= 1:
                finish(c - 1)
        finish(nch - 1)
        sd[nch - 2].wait()
        sd[nch - 1].wait()

    return emb


def kernel(x, table, pe):
    b, s = x.shape
    vocab, d = table.shape
    xf = x.reshape(-1).astype(jnp.int32)
    out = _build(b * s, s, vocab, d)(xf, table, pe)
    return out.reshape(b, s, d)


# speedup vs baseline: 1.1253x; 1.0331x over previous
import functools

import jax
import jax.numpy as jnp
from jax import lax
from jax.experimental import pallas as pl
from jax.experimental.pallas import tpu as pltpu
from jax.experimental.pallas import tpu_sc as plsc

NC, NS = 2, 16
NW = NC * NS
LANES = 16
CHUNK = 16


@functools.cache
def _build(nb, seq, vocab, d):
    ppw = seq // NW
    ncp = ppw // CHUNK
    nun = ncp * nb
    vecs = d // LANES
    mesh = plsc.VectorSubcoreMesh(core_axis_name="c", subcore_axis_name="s")

    @functools.partial(
        pl.kernel,
        mesh=mesh,
        out_type=jax.ShapeDtypeStruct((nb * seq, d), jnp.float32),
        scratch_types=[
            pltpu.VMEM((nb * ppw,), jnp.int32),
            pltpu.VMEM((CHUNK, d), jnp.float32),
            pltpu.VMEM((CHUNK, d), jnp.float32),
            pltpu.VMEM((CHUNK, d), jnp.float32),
            pltpu.VMEM((CHUNK, d), jnp.float32),
            pltpu.SemaphoreType.DMA,
            pltpu.SemaphoreType.DMA,
            pltpu.SemaphoreType.DMA,
            pltpu.SemaphoreType.DMA,
            pltpu.SemaphoreType.DMA,
            pltpu.SemaphoreType.DMA,
        ],
    )
    def emb(x_hbm, table_hbm, pe_hbm, out_hbm,
            idx_v, g0, g1, p0, p1, sg0, sg1, sp0, sp1, ss0, ss1):
        wid = lax.axis_index("s") * NC + lax.axis_index("c")
        pbase = wid * ppw
        gbuf = [g0, g1]
        pbuf = [p0, p1]
        sg = [sg0, sg1]
        sp = [sp0, sp1]
        ss = [ss0, ss1]
        gd = [None] * nun
        pd = [None] * ncp
        sd = [None] * nun

        for b in range(nb):
            pltpu.sync_copy(x_hbm.at[pl.ds(b * seq + pbase, ppw)],
                            idx_v.at[pl.ds(b * ppw, ppw)])

        def load_pe(c):
            pd[c] = pltpu.async_copy(
                pe_hbm.at[pl.ds(pbase + c * CHUNK, CHUNK)],
                pbuf[c % 2], sp[c % 2])

        def start(u):
            c, b = divmod(u, nb)
            gd[u] = pltpu.async_copy(
                table_hbm.at[idx_v.at[pl.ds(b * ppw + c * CHUNK, CHUNK)]],
                gbuf[u % 2], sg[u % 2])

        def finish(u):
            c, b = divmod(u, nb)
            gd[u].wait()
            if b == 0:
                pd[c].wait()
            g, p = gbuf[u % 2], pbuf[c % 2]

            def row(r, _):
                for j in range(vecs):
                    plsc.addupdate(g.at[r, pl.ds(j * LANES, LANES)],
                                   p[r, pl.ds(j * LANES, LANES)])
                return 0

            lax.fori_loop(0, CHUNK, row, 0, unroll=False)
            sd[u] = pltpu.async_copy(
                g, out_hbm.at[pl.ds(b * seq + pbase + c * CHUNK, CHUNK)],
                ss[u % 2])
            if b == nb - 1 and c + 2 < ncp:
                load_pe(c + 2)

        load_pe(0)
        if ncp > 1:
            load_pe(1)
        for u in range(nun):
            if u >= 2:
                sd[u - 2].wait()
            start(u)
            if u >= 1:
                finish(u - 1)
        finish(nun - 1)
        sd[nun - 2].wait()
        sd[nun - 1].wait()

    return emb


def kernel(x, table, pe):
    b, s = x.shape
    vocab, d = table.shape
    xf = x.reshape(-1).astype(jnp.int32)
    out = _build(b, s, vocab, d)(xf, table, pe)
    return out.reshape(b, s, d)


# speedup vs baseline: 1.3191x; 1.1722x over previous
import functools

import jax
import jax.numpy as jnp
from jax import lax
from jax.experimental import pallas as pl
from jax.experimental.pallas import tpu as pltpu
from jax.experimental.pallas import tpu_sc as plsc

NC, NS = 2, 16
NW = NC * NS
LANES = 16
CHUNK = 16


@functools.cache
def _build(nb, seq, vocab, d):
    ppw = seq // NW
    ncp = ppw // CHUNK
    nun = ncp * nb
    vecs = d // LANES
    mesh = plsc.VectorSubcoreMesh(core_axis_name="c", subcore_axis_name="s")

    @functools.partial(
        pl.kernel,
        mesh=mesh,
        out_type=jax.ShapeDtypeStruct((nb * seq, d), jnp.float32),
        scratch_types=[
            pltpu.VMEM((nb * ppw,), jnp.int32),
            pltpu.VMEM((CHUNK, d), jnp.float32),
            pltpu.VMEM((CHUNK, d), jnp.float32),
            pltpu.VMEM((CHUNK, d), jnp.float32),
            pltpu.VMEM((CHUNK, d), jnp.float32),
            pltpu.SemaphoreType.DMA,
            pltpu.SemaphoreType.DMA,
            pltpu.SemaphoreType.DMA,
            pltpu.SemaphoreType.DMA,
            pltpu.SemaphoreType.DMA,
            pltpu.SemaphoreType.DMA,
        ],
    )
    def emb(x_hbm, table_hbm, pe_hbm, out_hbm,
            idx_v, g0, g1, p0, p1, sg0, sg1, sp0, sp1, ss0, ss1):
        wid = lax.axis_index("s") * NC + lax.axis_index("c")
        pbase = wid * ppw
        gbuf = [g0, g1]
        pbuf = [p0, p1]
        sg = [sg0, sg1]
        sp = [sp0, sp1]
        ss = [ss0, ss1]
        gd = [None] * nun
        pd = [None] * ncp
        sd = [None] * nun

        for b in range(nb):
            pltpu.sync_copy(x_hbm.at[pl.ds(b * seq + pbase, ppw)],
                            idx_v.at[pl.ds(b * ppw, ppw)])

        def load_pe(c):
            pd[c] = pltpu.async_copy(
                pe_hbm.at[pl.ds(pbase + c * CHUNK, CHUNK)],
                pbuf[c % 2], sp[c % 2])

        def start(u):
            c, b = divmod(u, nb)
            gd[u] = pltpu.async_copy(
                table_hbm.at[idx_v.at[pl.ds(b * ppw + c * CHUNK, CHUNK)]],
                gbuf[u % 2], sg[u % 2])

        def finish(u):
            c, b = divmod(u, nb)
            gd[u].wait()
            if b == 0:
                pd[c].wait()
            g, p = gbuf[u % 2], pbuf[c % 2]

            shift = vecs.bit_length() - 1

            @plsc.parallel_loop(0, CHUNK * vecs, unroll=4)
            def vec(i):
                r = jax.lax.shift_right_logical(i, shift)
                col = (i & (vecs - 1)) * LANES
                plsc.addupdate(g.at[r, pl.ds(col, LANES)],
                               p[r, pl.ds(col, LANES)])
            sd[u] = pltpu.async_copy(
                g, out_hbm.at[pl.ds(b * seq + pbase + c * CHUNK, CHUNK)],
                ss[u % 2])
            if b == nb - 1 and c + 2 < ncp:
                load_pe(c + 2)

        load_pe(0)
        if ncp > 1:
            load_pe(1)
        for u in range(nun):
            if u >= 2:
                sd[u - 2].wait()
            start(u)
            if u >= 1:
                finish(u - 1)
        finish(nun - 1)
        sd[nun - 2].wait()
        sd[nun - 1].wait()

    return emb


def kernel(x, table, pe):
    b, s = x.shape
    vocab, d = table.shape
    xf = x.reshape(-1).astype(jnp.int32)
    out = _build(b, s, vocab, d)(xf, table, pe)
    return out.reshape(b, s, d)


# speedup vs baseline: 1.3522x; 1.0251x over previous
import functools

import jax
import jax.numpy as jnp
from jax import lax
from jax.experimental import pallas as pl
from jax.experimental.pallas import tpu as pltpu
from jax.experimental.pallas import tpu_sc as plsc

NC, NS = 2, 16
NW = NC * NS
LANES = 16
CHUNK = 16
GBUF = 4


@functools.cache
def _build(nb, seq, vocab, d):
    ppw = seq // NW
    ncp = ppw // CHUNK
    nun = ncp * nb
    vecs = d // LANES
    assert vecs & (vecs - 1) == 0
    shift = vecs.bit_length() - 1
    mesh = plsc.VectorSubcoreMesh(core_axis_name="c", subcore_axis_name="s")

    @functools.partial(
        pl.kernel,
        mesh=mesh,
        out_type=jax.ShapeDtypeStruct((nb * seq, d), jnp.float32),
        scratch_types=[
            pltpu.VMEM((nb * ppw,), jnp.int32),
            *[pltpu.VMEM((CHUNK, d), jnp.float32) for _ in range(GBUF + 2)],
            *[pltpu.SemaphoreType.DMA for _ in range(2 * GBUF + 2)],
        ],
    )
    def emb(x_hbm, table_hbm, pe_hbm, out_hbm, idx_v, *bufs_sems):
        gbuf = list(bufs_sems[:GBUF])
        pbuf = list(bufs_sems[GBUF:GBUF + 2])
        sems = bufs_sems[GBUF + 2:]
        sg = list(sems[:GBUF])
        ss = list(sems[GBUF:2 * GBUF])
        sp = list(sems[2 * GBUF:])
        wid = lax.axis_index("s") * NC + lax.axis_index("c")
        pbase = wid * ppw
        gd = [None] * nun
        pd = [None] * ncp
        sd = [None] * nun

        for b in range(nb):
            pltpu.sync_copy(x_hbm.at[pl.ds(b * seq + pbase, ppw)],
                            idx_v.at[pl.ds(b * ppw, ppw)])

        def load_pe(c):
            pd[c] = pltpu.async_copy(
                pe_hbm.at[pl.ds(pbase + c * CHUNK, CHUNK)],
                pbuf[c % 2], sp[c % 2])

        def start(u):
            c, b = divmod(u, nb)
            gd[u] = pltpu.async_copy(
                table_hbm.at[idx_v.at[pl.ds(b * ppw + c * CHUNK, CHUNK)]],
                gbuf[u % GBUF], sg[u % GBUF])

        def finish(u):
            c, b = divmod(u, nb)
            gd[u].wait()
            if b == 0:
                pd[c].wait()
            g, p = gbuf[u % GBUF], pbuf[c % 2]

            @plsc.parallel_loop(0, CHUNK * vecs, unroll=8)
            def vec(i):
                r = jax.lax.shift_right_logical(i, shift)
                col = (i & (vecs - 1)) * LANES
                plsc.addupdate(g.at[r, pl.ds(col, LANES)],
                               p[r, pl.ds(col, LANES)])

            sd[u] = pltpu.async_copy(
                g, out_hbm.at[pl.ds(b * seq + pbase + c * CHUNK, CHUNK)],
                ss[u % GBUF])
            if b == nb - 1 and c + 2 < ncp:
                load_pe(c + 2)

        load_pe(0)
        if ncp > 1:
            load_pe(1)
        lag = GBUF - 1
        for u in range(nun):
            if u >= GBUF:
                sd[u - GBUF].wait()
            start(u)
            if u >= lag:
                finish(u - lag)
        for u in range(nun - lag, nun):
            finish(u)
        for u in range(nun - GBUF, nun):
            sd[u].wait()

    return emb


def kernel(x, table, pe):
    b, s = x.shape
    vocab, d = table.shape
    xf = x.reshape(-1).astype(jnp.int32)
    out = _build(b, s, vocab, d)(xf, table, pe)
    return out.reshape(b, s, d)


# speedup vs baseline: 1.3808x; 1.0211x over previous
import functools

import jax
import jax.numpy as jnp
from jax import lax
from jax.experimental import pallas as pl
from jax.experimental.pallas import tpu as pltpu
from jax.experimental.pallas import tpu_sc as plsc

NC, NS = 2, 16
NW = NC * NS
LANES = 16
CHUNK = 16
GBUF = 5


@functools.cache
def _build(nb, seq, vocab, d):
    ppw = seq // NW
    ncp = ppw // CHUNK
    nun = ncp * nb
    vecs = d // LANES
    assert vecs & (vecs - 1) == 0
    shift = vecs.bit_length() - 1
    mesh = plsc.VectorSubcoreMesh(core_axis_name="c", subcore_axis_name="s")

    @functools.partial(
        pl.kernel,
        mesh=mesh,
        out_type=jax.ShapeDtypeStruct((nb * seq, d), jnp.float32),
        scratch_types=[
            pltpu.VMEM((nb, ppw), jnp.int32),
            *[pltpu.VMEM((CHUNK, d), jnp.float32) for _ in range(GBUF + 2)],
            *[pltpu.SemaphoreType.DMA for _ in range(2 * GBUF + 3)],
        ],
    )
    def emb(x_hbm, table_hbm, pe_hbm, out_hbm, idx_v, *bufs_sems):
        gbuf = list(bufs_sems[:GBUF])
        pbuf = list(bufs_sems[GBUF:GBUF + 2])
        sems = bufs_sems[GBUF + 2:]
        sg = list(sems[:GBUF])
        ss = list(sems[GBUF:2 * GBUF])
        sp = list(sems[2 * GBUF:2 * GBUF + 2])
        si = sems[2 * GBUF + 2]
        wid = lax.axis_index("s") * NC + lax.axis_index("c")
        pbase = wid * ppw
        gd = [None] * nun
        pd = [None] * ncp
        sd = [None] * nun

        idx_copy = pltpu.async_copy(
            x_hbm.at[:, pl.ds(pbase, ppw)], idx_v, si)

        def load_pe(c):
            pd[c] = pltpu.async_copy(
                pe_hbm.at[pl.ds(pbase + c * CHUNK, CHUNK)],
                pbuf[c % 2], sp[c % 2])

        def start(u):
            c, b = divmod(u, nb)
            gd[u] = pltpu.async_copy(
                table_hbm.at[idx_v.at[b, pl.ds(c * CHUNK, CHUNK)]],
                gbuf[u % GBUF], sg[u % GBUF])

        def finish(u):
            c, b = divmod(u, nb)
            gd[u].wait()
            if b == 0:
                pd[c].wait()
            g, p = gbuf[u % GBUF], pbuf[c % 2]

            @plsc.parallel_loop(0, CHUNK * vecs, unroll=8)
            def vec(i):
                r = jax.lax.shift_right_logical(i, shift)
                col = (i & (vecs - 1)) * LANES
                plsc.addupdate(g.at[r, pl.ds(col, LANES)],
                               p[r, pl.ds(col, LANES)])

            sd[u] = pltpu.async_copy(
                g, out_hbm.at[pl.ds(b * seq + pbase + c * CHUNK, CHUNK)],
                ss[u % GBUF])
            if b == nb - 1 and c + 2 < ncp:
                load_pe(c + 2)

        load_pe(0)
        if ncp > 1:
            load_pe(1)
        idx_copy.wait()
        lag = GBUF - 1
        for u in range(nun):
            if u >= GBUF:
                sd[u - GBUF].wait()
            start(u)
            if u >= lag:
                finish(u - lag)
        for u in range(nun - lag, nun):
            finish(u)
        for u in range(nun - GBUF, nun):
            sd[u].wait()

    return emb


def kernel(x, table, pe):
    b, s = x.shape
    vocab, d = table.shape
    out = _build(b, s, vocab, d)(x.astype(jnp.int32), table, pe)
    return out.reshape(b, s, d)
